# baseline (speedup 1.0000x reference)
import functools

import jax
import jax.numpy as jnp
from jax import lax
from jax.experimental import pallas as pl
from jax.experimental.pallas import tpu as pltpu
from jax.experimental.pallas import tpu_sc as plsc

N = 10000
F = 128
NC = 2
NS = 16
NW = NC * NS
CW = 128
ACC_ROWS = 10240
ROWS_PER_TILE = ACC_ROWS // NS

_mesh = functools.partial(
    plsc.VectorSubcoreMesh, core_axis_name="c", subcore_axis_name="s")


def _wid():
    return lax.axis_index("s") * NC + lax.axis_index("c")


def _zero_vmem_2d(buf, rows, cols):
    z = jnp.zeros((16,), jnp.float32)

    def body(i, _):
        for k in range(cols // 16):
            buf[i, pl.ds(k * 16, 16)] = z
        return 0

    lax.fori_loop(0, rows, body, 0)


def _zero_vmem_1d(buf, n):
    z = jnp.zeros((16,), jnp.float32)

    def body(i, _):
        buf[pl.ds(i * 16, 16)] = z
        return 0

    lax.fori_loop(0, n // 16, body, 0)


def _ring_gather_scatter(src_hbm, acc, src_v, dst_v, bufs, gsems, ssems,
                         nchunk, lk=None, nsplit=1):
    r = len(gsems)
    lk = r // 2 if lk is None else lk
    ngroups = nchunk // r
    sub = bufs.shape[1] // nsplit

    def gather(j, b):
        pltpu.async_copy(src_hbm.at[src_v(j)], bufs.at[b], gsems[b])

    def wait_gather(j, b):
        pltpu.make_async_copy(
            src_hbm.at[src_v(j)], bufs.at[b], gsems[b]).wait()

    def scatter(j, b):
        for p in range(nsplit):
            pltpu.async_copy(
                bufs.at[b, pl.ds(p * sub, sub)], acc.at[dst_v(j, p)],
                ssems[b], add=True)

    def wait_scatter(j, b):
        for p in range(nsplit):
            pltpu.make_async_copy(
                bufs.at[b, pl.ds(p * sub, sub)], acc.at[dst_v(j, p)],
                ssems[b]).wait()

    for b in range(lk):
        gather(b, b)

    def group(i, _):
        for b in range(r):
            j = i * r + b
            bw = (b + lk) % r
            wait_gather(j, b)
            scatter(j, b)
            if b >= r - lk:
                wait_scatter(j - (r - lk), bw)

                @pl.when(i < ngroups - 1)
                def _():
                    gather(j + lk, bw)
            else:
                @pl.when(i > 0)
                def _():
                    wait_scatter(j - (r - lk), bw)

                gather(j + lk, bw)
        return 0

    lax.fori_loop(0, ngroups, group, 0)
    for k in range(nchunk - (r - lk), nchunk):
        wait_scatter(k, k % r)


def _make_deg_kernel(nchunk):
    nsem = 8

    @functools.partial(
        pl.kernel,
        out_type=jax.ShapeDtypeStruct((NC, ACC_ROWS), jnp.float32),
        mesh=_mesh(),
        scratch_types=[
            pltpu.VMEM_SHARED((ACC_ROWS,), jnp.float32),
            pltpu.VMEM((nchunk, CW), jnp.int32),
            pltpu.VMEM((CW,), jnp.float32),
            pltpu.VMEM((ROWS_PER_TILE,), jnp.float32),
        ] + [pltpu.SemaphoreType.DMA] * nsem,
    )
    def deg_kernel(dst_hbm, out_hbm, acc, dst_v, ones_v, zb, *sems):
        c = lax.axis_index("c")
        s = lax.axis_index("s")
        _zero_vmem_1d(zb, ROWS_PER_TILE)
        pltpu.sync_copy(zb, acc.at[pl.ds(s * ROWS_PER_TILE, ROWS_PER_TILE)])

        one = jnp.ones((16,), jnp.float32)

        def setone(i, _):
            ones_v[pl.ds(i * 16, 16)] = one
            return 0

        lax.fori_loop(0, CW // 16, setone, 0)
        pltpu.sync_copy(dst_hbm.at[_wid()], dst_v)
        plsc.subcore_barrier()

        def group(i, _):
            for b in range(nsem):
                j = i * nsem + b

                @pl.when(i > 0)
                def _():
                    pltpu.make_async_copy(
                        ones_v, acc.at[dst_v.at[j - nsem]], sems[b]).wait()

                pltpu.async_copy(
                    ones_v, acc.at[dst_v.at[j]], sems[b], add=True)
            return 0

        lax.fori_loop(0, nchunk // nsem, group, 0)
        for b in range(nsem):
            pltpu.make_async_copy(
                ones_v, acc.at[dst_v.at[nchunk - nsem + b]], sems[b]).wait()
        plsc.subcore_barrier()
        pltpu.sync_copy(
            acc.at[pl.ds(s * ROWS_PER_TILE, ROWS_PER_TILE)],
            out_hbm.at[c, pl.ds(s * ROWS_PER_TILE, ROWS_PER_TILE)])

    return deg_kernel


def _make_seg_kernel(nchunk, cw):
    nbuf = 6

    @functools.partial(
        pl.kernel,
        out_type=jax.ShapeDtypeStruct((NC, ACC_ROWS, F), jnp.float32),
        mesh=_mesh(),
        scratch_types=[
            pltpu.VMEM_SHARED((ACC_ROWS, F), jnp.float32),
            pltpu.VMEM((nchunk * cw,), jnp.int32),
            pltpu.VMEM((nchunk * cw,), jnp.int32),
            pltpu.VMEM((nbuf, cw, F), jnp.float32),
        ] + [pltpu.SemaphoreType.DMA] * (2 * nbuf),
    )
    def seg_kernel(y1s_hbm, src_hbm, dst_hbm, out_hbm,
                   acc, src_v, dst_v, rows_v, *sems):
        c = lax.axis_index("c")
        s = lax.axis_index("s")

        z = jnp.zeros((16,), jnp.float32)

        def zrow(i, _):
            for k in range(F // 16):
                rows_v[0, i, pl.ds(k * 16, 16)] = z
            return 0

        lax.fori_loop(0, cw, zrow, 0)

        def zcopy(i, _):
            pltpu.sync_copy(
                rows_v.at[0],
                acc.at[pl.ds(s * ROWS_PER_TILE + i * cw, cw)])
            return 0

        lax.fori_loop(0, ROWS_PER_TILE // cw, zcopy, 0)
        pltpu.sync_copy(src_hbm.at[_wid()], src_v)
        pltpu.sync_copy(dst_hbm.at[_wid()], dst_v)
        plsc.subcore_barrier()

        _ring_gather_scatter(
            y1s_hbm, acc,
            lambda j: src_v.at[pl.ds(j * cw, cw)],
            lambda j, p: dst_v[pl.ds(j * cw + p * 16, 16)],
            rows_v, sems[:nbuf], sems[nbuf:], nchunk, nsplit=cw // 16)
        plsc.subcore_barrier()
        pltpu.sync_copy(
            acc.at[pl.ds(s * ROWS_PER_TILE, ROWS_PER_TILE)],
            out_hbm.at[c, pl.ds(s * ROWS_PER_TILE, ROWS_PER_TILE)])

    return seg_kernel


def _make_scalar_seg_kernel(nchunk):
    nbuf = 8

    @functools.partial(
        pl.kernel,
        out_type=jax.ShapeDtypeStruct((NC, ACC_ROWS), jnp.float32),
        mesh=_mesh(),
        scratch_types=[
            pltpu.VMEM_SHARED((ACC_ROWS,), jnp.float32),
            pltpu.VMEM_SHARED((ACC_ROWS,), jnp.float32),
            pltpu.VMEM((nchunk * CW,), jnp.int32),
            pltpu.VMEM((nchunk, CW), jnp.int32),
            pltpu.VMEM((nbuf, CW), jnp.float32),
            pltpu.VMEM((ROWS_PER_TILE,), jnp.float32),
        ] + [pltpu.SemaphoreType.DMA] * (2 * nbuf),
    )
    def sseg_kernel(vs_hbm, src_hbm, dst_hbm, out_hbm,
                    acc, vs_sp, src_v, dst_v, gbuf, zb, *sems):
        c = lax.axis_index("c")
        s = lax.axis_index("s")
        _zero_vmem_1d(zb, ROWS_PER_TILE)
        pltpu.sync_copy(zb, acc.at[pl.ds(s * ROWS_PER_TILE, ROWS_PER_TILE)])

        @pl.when(s == 0)
        def _():
            pltpu.sync_copy(vs_hbm.at[0], vs_sp)

        pltpu.sync_copy(src_hbm.at[_wid()], src_v)
        pltpu.sync_copy(dst_hbm.at[_wid()], dst_v)
        plsc.subcore_barrier()

        _ring_gather_scatter(
            vs_sp, acc,
            lambda j: src_v.at[pl.ds(j * CW, CW)],
            lambda j, p: dst_v.at[j],
            gbuf, sems[:nbuf], sems[nbuf:], nchunk)
        plsc.subcore_barrier()
        pltpu.sync_copy(
            acc.at[pl.ds(s * ROWS_PER_TILE, ROWS_PER_TILE)],
            out_hbm.at[c, pl.ds(s * ROWS_PER_TILE, ROWS_PER_TILE)])

    return sseg_kernel


_B = 2048


def _dis_block(cnt_ref):
    j = pl.program_id(0)
    cnt = cnt_ref[:, pl.ds(j * _B, _B)]
    return lax.rsqrt(cnt[0] + cnt[1] + 1.0)


def _tc1_body(cnt_ref, x_ref, w1_ref, y1s_ref):
    dis = _dis_block(cnt_ref)
    y1 = jnp.dot(x_ref[...], w1_ref[...], preferred_element_type=jnp.float32)
    y1s_ref[...] = y1 * dis[:, None]


def _tc2_body(cnt_ref, acc_ref, y1s_ref, b1_ref, w2_ref, vs_ref):
    dis = _dis_block(cnt_ref)
    tot = acc_ref[0] + acc_ref[1] + y1s_ref[...]
    h = jnp.maximum(tot * dis[:, None] + b1_ref[...][None, :], 0.0)
    v = jnp.dot(h, w2_ref[...], preferred_element_type=jnp.float32)
    vs_ref[...] = v * dis[:, None]


def _tc3_body(cnt_ref, acc2_ref, vs_ref, b2_ref, out_ref):
    j = pl.program_id(0)
    dis = _dis_block(cnt_ref)
    acc2 = acc2_ref[:, pl.ds(j * _B, _B)]
    tot = acc2[0] + acc2[1] + vs_ref[...][:, 0]
    out_ref[...] = (tot * dis + b2_ref[0])[:, None]


def _pad_indices(src, dst, nchunk, cw):
    e = src.shape[0]
    npad = NW * nchunk * cw - e
    rng = jnp.arange(npad, dtype=jnp.int32)
    pad_src = rng % N
    pad_dst = N + rng % (ACC_ROWS - N)
    srcp = jnp.concatenate([src, pad_src]).reshape(NW, nchunk, cw)
    dstp = jnp.concatenate([dst, pad_dst]).reshape(NW, nchunk, cw)
    return srcp, dstp


def kernel(x, edge_index, W1, b1, W2, b2):
    e = edge_index.shape[1]
    src = edge_index[0].astype(jnp.int32)
    dst = edge_index[1].astype(jnp.int32)

    nchunk = (-(-e // (NW * CW)) + 7) // 8 * 8
    srcp, dstp = _pad_indices(src, dst, nchunk, CW)
    srcp = srcp.reshape(NW, nchunk * CW)
    cw1 = 32
    nchunk1 = (-(-e // (NW * cw1)) + 5) // 6 * 6
    srcp1, dstp1 = _pad_indices(src, dst, nchunk1, cw1)
    srcp1 = srcp1.reshape(NW, nchunk1 * cw1)
    dstp1 = dstp1.reshape(NW, nchunk1 * cw1)

    cnt = _make_deg_kernel(nchunk)(dstp)

    grid = pl.cdiv(N, _B)
    y1s = pl.pallas_call(
        _tc1_body,
        grid=(grid,),
        in_specs=[
            pl.BlockSpec((NC, ACC_ROWS), lambda j: (0, 0)),
            pl.BlockSpec((_B, F), lambda j: (j, 0)),
            pl.BlockSpec((F, F), lambda j: (0, 0)),
        ],
        out_specs=pl.BlockSpec((_B, F), lambda j: (j, 0)),
        out_shape=jax.ShapeDtypeStruct((N, F), jnp.float32),
    )(cnt, x, W1)

    acc1 = _make_seg_kernel(nchunk1, cw1)(y1s, srcp1, dstp1)

    vs2d = pl.pallas_call(
        _tc2_body,
        grid=(grid,),
        in_specs=[
            pl.BlockSpec((NC, ACC_ROWS), lambda j: (0, 0)),
            pl.BlockSpec((NC, _B, F), lambda j: (0, j, 0)),
            pl.BlockSpec((_B, F), lambda j: (j, 0)),
            pl.BlockSpec((F,), lambda j: (0,)),
            pl.BlockSpec((F, 1), lambda j: (0, 0)),
        ],
        out_specs=pl.BlockSpec((_B, 1), lambda j: (j, 0)),
        out_shape=jax.ShapeDtypeStruct((N, 1), jnp.float32),
    )(cnt, acc1, y1s, b1, W2)

    vs = jnp.pad(vs2d.reshape(1, N), ((0, 0), (0, ACC_ROWS - N)))
    acc2 = _make_scalar_seg_kernel(nchunk)(vs, srcp, dstp)

    out = pl.pallas_call(
        _tc3_body,
        grid=(grid,),
        in_specs=[
            pl.BlockSpec((NC, ACC_ROWS), lambda j: (0, 0)),
            pl.BlockSpec((NC, ACC_ROWS), lambda j: (0, 0)),
            pl.BlockSpec((_B, 1), lambda j: (j, 0)),
            pl.BlockSpec((1,), lambda j: (0,)),
        ],
        out_specs=pl.BlockSpec((_B, 1), lambda j: (j, 0)),
        out_shape=jax.ShapeDtypeStruct((N, 1), jnp.float32),
    )(cnt, acc2, vs2d, b2)

    return out


# speedup vs baseline: 1.0819x; 1.0819x over previous
import functools

import jax
import jax.numpy as jnp
from jax import lax
from jax.experimental import pallas as pl
from jax.experimental.pallas import tpu as pltpu
from jax.experimental.pallas import tpu_sc as plsc

N = 10000
F = 128
NC = 2
NS = 16
NW = NC * NS
CW = 128
ACC_ROWS = 10240
ROWS_PER_TILE = ACC_ROWS // NS

_mesh = functools.partial(
    plsc.VectorSubcoreMesh, core_axis_name="c", subcore_axis_name="s")


def _wid():
    return lax.axis_index("s") * NC + lax.axis_index("c")


def _zero_vmem_2d(buf, rows, cols):
    z = jnp.zeros((16,), jnp.float32)

    def body(i, _):
        for k in range(cols // 16):
            buf[i, pl.ds(k * 16, 16)] = z
        return 0

    lax.fori_loop(0, rows, body, 0)


def _zero_vmem_1d(buf, n):
    z = jnp.zeros((16,), jnp.float32)

    def body(i, _):
        buf[pl.ds(i * 16, 16)] = z
        return 0

    lax.fori_loop(0, n // 16, body, 0)


def _ring_gather_scatter(src_hbm, acc, src_v, dst_v, bufs, gsems, ssems,
                         nchunk, lk=None, nsplit=1):
    r = len(gsems)
    lk = r // 2 if lk is None else lk
    ngroups = nchunk // r
    sub = bufs.shape[1] // nsplit

    def gather(j, b):
        pltpu.async_copy(src_hbm.at[src_v(j)], bufs.at[b], gsems[b])

    def wait_gather(j, b):
        pltpu.make_async_copy(
            src_hbm.at[src_v(j)], bufs.at[b], gsems[b]).wait()

    def scatter(j, b):
        for p in range(nsplit):
            pltpu.async_copy(
                bufs.at[b, pl.ds(p * sub, sub)], acc.at[dst_v(j, p)],
                ssems[b], add=True)

    def wait_scatter(j, b):
        for p in range(nsplit):
            pltpu.make_async_copy(
                bufs.at[b, pl.ds(p * sub, sub)], acc.at[dst_v(j, p)],
                ssems[b]).wait()

    for b in range(lk):
        gather(b, b)

    def group(i, _):
        for b in range(r):
            j = i * r + b
            bw = (b + lk) % r
            wait_gather(j, b)
            scatter(j, b)
            if b >= r - lk:
                wait_scatter(j - (r - lk), bw)

                @pl.when(i < ngroups - 1)
                def _():
                    gather(j + lk, bw)
            else:
                @pl.when(i > 0)
                def _():
                    wait_scatter(j - (r - lk), bw)

                gather(j + lk, bw)
        return 0

    lax.fori_loop(0, ngroups, group, 0)
    for k in range(nchunk - (r - lk), nchunk):
        wait_scatter(k, k % r)


def _make_deg_kernel(nchunk):
    nsem = 8

    @functools.partial(
        pl.kernel,
        out_type=jax.ShapeDtypeStruct((NC, ACC_ROWS), jnp.float32),
        mesh=_mesh(),
        scratch_types=[
            pltpu.VMEM_SHARED((ACC_ROWS,), jnp.float32),
            pltpu.VMEM((nchunk, CW), jnp.int32),
            pltpu.VMEM((CW,), jnp.float32),
            pltpu.VMEM((ROWS_PER_TILE,), jnp.float32),
        ] + [pltpu.SemaphoreType.DMA] * nsem,
    )
    def deg_kernel(dst_hbm, out_hbm, acc, dst_v, ones_v, zb, *sems):
        c = lax.axis_index("c")
        s = lax.axis_index("s")
        _zero_vmem_1d(zb, ROWS_PER_TILE)
        pltpu.sync_copy(zb, acc.at[pl.ds(s * ROWS_PER_TILE, ROWS_PER_TILE)])

        one = jnp.ones((16,), jnp.float32)

        def setone(i, _):
            ones_v[pl.ds(i * 16, 16)] = one
            return 0

        lax.fori_loop(0, CW // 16, setone, 0)
        pltpu.sync_copy(dst_hbm.at[_wid()], dst_v)
        plsc.subcore_barrier()

        def group(i, _):
            for b in range(nsem):
                j = i * nsem + b

                @pl.when(i > 0)
                def _():
                    pltpu.make_async_copy(
                        ones_v, acc.at[dst_v.at[j - nsem]], sems[b]).wait()

                pltpu.async_copy(
                    ones_v, acc.at[dst_v.at[j]], sems[b], add=True)
            return 0

        lax.fori_loop(0, nchunk // nsem, group, 0)
        for b in range(nsem):
            pltpu.make_async_copy(
                ones_v, acc.at[dst_v.at[nchunk - nsem + b]], sems[b]).wait()
        plsc.subcore_barrier()
        pltpu.sync_copy(
            acc.at[pl.ds(s * ROWS_PER_TILE, ROWS_PER_TILE)],
            out_hbm.at[c, pl.ds(s * ROWS_PER_TILE, ROWS_PER_TILE)])

    return deg_kernel


def _make_seg_kernel(nchunk, cw):
    nbuf = 12

    @functools.partial(
        pl.kernel,
        out_type=jax.ShapeDtypeStruct((NC, ACC_ROWS, F), jnp.float32),
        mesh=_mesh(),
        scratch_types=[
            pltpu.VMEM_SHARED((ACC_ROWS, F), jnp.float32),
            pltpu.VMEM((nchunk * cw,), jnp.int32),
            pltpu.VMEM((nchunk * cw,), jnp.int32),
            pltpu.VMEM((nbuf, cw, F), jnp.float32),
        ] + [pltpu.SemaphoreType.DMA] * (2 * nbuf),
    )
    def seg_kernel(y1s_hbm, src_hbm, dst_hbm, out_hbm,
                   acc, src_v, dst_v, rows_v, *sems):
        c = lax.axis_index("c")
        s = lax.axis_index("s")

        z = jnp.zeros((16,), jnp.float32)

        def zrow(i, _):
            for k in range(F // 16):
                rows_v[0, i, pl.ds(k * 16, 16)] = z
            return 0

        lax.fori_loop(0, cw, zrow, 0)

        def zcopy(i, _):
            pltpu.sync_copy(
                rows_v.at[0],
                acc.at[pl.ds(s * ROWS_PER_TILE + i * cw, cw)])
            return 0

        lax.fori_loop(0, ROWS_PER_TILE // cw, zcopy, 0)
        pltpu.sync_copy(src_hbm.at[_wid()], src_v)
        pltpu.sync_copy(dst_hbm.at[_wid()], dst_v)
        plsc.subcore_barrier()

        _ring_gather_scatter(
            y1s_hbm, acc,
            lambda j: src_v[pl.ds(j * cw, cw)],
            lambda j, p: dst_v[pl.ds(j * cw, cw)],
            rows_v, sems[:nbuf], sems[nbuf:], nchunk, lk=7)
        plsc.subcore_barrier()
        pltpu.sync_copy(
            acc.at[pl.ds(s * ROWS_PER_TILE, ROWS_PER_TILE)],
            out_hbm.at[c, pl.ds(s * ROWS_PER_TILE, ROWS_PER_TILE)])

    return seg_kernel


def _make_scalar_seg_kernel(nchunk):
    nbuf = 8

    @functools.partial(
        pl.kernel,
        out_type=jax.ShapeDtypeStruct((NC, ACC_ROWS), jnp.float32),
        mesh=_mesh(),
        scratch_types=[
            pltpu.VMEM_SHARED((ACC_ROWS,), jnp.float32),
            pltpu.VMEM_SHARED((ACC_ROWS,), jnp.float32),
            pltpu.VMEM((nchunk * CW,), jnp.int32),
            pltpu.VMEM((nchunk, CW), jnp.int32),
            pltpu.VMEM((nbuf, CW), jnp.float32),
            pltpu.VMEM((ROWS_PER_TILE,), jnp.float32),
        ] + [pltpu.SemaphoreType.DMA] * (2 * nbuf),
    )
    def sseg_kernel(vs_hbm, src_hbm, dst_hbm, out_hbm,
                    acc, vs_sp, src_v, dst_v, gbuf, zb, *sems):
        c = lax.axis_index("c")
        s = lax.axis_index("s")
        _zero_vmem_1d(zb, ROWS_PER_TILE)
        pltpu.sync_copy(zb, acc.at[pl.ds(s * ROWS_PER_TILE, ROWS_PER_TILE)])

        @pl.when(s == 0)
        def _():
            pltpu.sync_copy(vs_hbm.at[0], vs_sp)

        pltpu.sync_copy(src_hbm.at[_wid()], src_v)
        pltpu.sync_copy(dst_hbm.at[_wid()], dst_v)
        plsc.subcore_barrier()

        _ring_gather_scatter(
            vs_sp, acc,
            lambda j: src_v.at[pl.ds(j * CW, CW)],
            lambda j, p: dst_v.at[j],
            gbuf, sems[:nbuf], sems[nbuf:], nchunk)
        plsc.subcore_barrier()
        pltpu.sync_copy(
            acc.at[pl.ds(s * ROWS_PER_TILE, ROWS_PER_TILE)],
            out_hbm.at[c, pl.ds(s * ROWS_PER_TILE, ROWS_PER_TILE)])

    return sseg_kernel


_B = 2048


def _dis_block(cnt_ref):
    j = pl.program_id(0)
    cnt = cnt_ref[:, pl.ds(j * _B, _B)]
    return lax.rsqrt(cnt[0] + cnt[1] + 1.0)


def _tc1_body(cnt_ref, x_ref, w1_ref, y1s_ref):
    dis = _dis_block(cnt_ref)
    y1 = jnp.dot(x_ref[...], w1_ref[...], preferred_element_type=jnp.float32)
    y1s_ref[...] = y1 * dis[:, None]


def _tc2_body(cnt_ref, acc_ref, y1s_ref, b1_ref, w2_ref, vs_ref):
    dis = _dis_block(cnt_ref)
    tot = acc_ref[0] + acc_ref[1] + y1s_ref[...]
    h = jnp.maximum(tot * dis[:, None] + b1_ref[...][None, :], 0.0)
    v = jnp.dot(h, w2_ref[...], preferred_element_type=jnp.float32)
    vs_ref[...] = v * dis[:, None]


def _tc3_body(cnt_ref, acc2_ref, vs_ref, b2_ref, out_ref):
    j = pl.program_id(0)
    dis = _dis_block(cnt_ref)
    acc2 = acc2_ref[:, pl.ds(j * _B, _B)]
    tot = acc2[0] + acc2[1] + vs_ref[...][:, 0]
    out_ref[...] = (tot * dis + b2_ref[0])[:, None]


def _pad_indices(src, dst, nchunk, cw):
    e = src.shape[0]
    npad = NW * nchunk * cw - e
    rng = jnp.arange(npad, dtype=jnp.int32)
    pad_src = rng % N
    pad_dst = N + rng % (ACC_ROWS - N)
    srcp = jnp.concatenate([src, pad_src]).reshape(NW, nchunk, cw)
    dstp = jnp.concatenate([dst, pad_dst]).reshape(NW, nchunk, cw)
    return srcp, dstp


def kernel(x, edge_index, W1, b1, W2, b2):
    e = edge_index.shape[1]
    src = edge_index[0].astype(jnp.int32)
    dst = edge_index[1].astype(jnp.int32)

    nchunk = (-(-e // (NW * CW)) + 7) // 8 * 8
    srcp, dstp = _pad_indices(src, dst, nchunk, CW)
    srcp = srcp.reshape(NW, nchunk * CW)
    cw1 = 16
    nchunk1 = (-(-e // (NW * cw1)) + 11) // 12 * 12
    srcp1, dstp1 = _pad_indices(src, dst, nchunk1, cw1)
    srcp1 = srcp1.reshape(NW, nchunk1 * cw1)
    dstp1 = dstp1.reshape(NW, nchunk1 * cw1)

    cnt = _make_deg_kernel(nchunk)(dstp)

    grid = pl.cdiv(N, _B)
    y1s = pl.pallas_call(
        _tc1_body,
        grid=(grid,),
        in_specs=[
            pl.BlockSpec((NC, ACC_ROWS), lambda j: (0, 0)),
            pl.BlockSpec((_B, F), lambda j: (j, 0)),
            pl.BlockSpec((F, F), lambda j: (0, 0)),
        ],
        out_specs=pl.BlockSpec((_B, F), lambda j: (j, 0)),
        out_shape=jax.ShapeDtypeStruct((N, F), jnp.float32),
    )(cnt, x, W1)

    acc1 = _make_seg_kernel(nchunk1, cw1)(y1s, srcp1, dstp1)

    vs2d = pl.pallas_call(
        _tc2_body,
        grid=(grid,),
        in_specs=[
            pl.BlockSpec((NC, ACC_ROWS), lambda j: (0, 0)),
            pl.BlockSpec((NC, _B, F), lambda j: (0, j, 0)),
            pl.BlockSpec((_B, F), lambda j: (j, 0)),
            pl.BlockSpec((F,), lambda j: (0,)),
            pl.BlockSpec((F, 1), lambda j: (0, 0)),
        ],
        out_specs=pl.BlockSpec((_B, 1), lambda j: (j, 0)),
        out_shape=jax.ShapeDtypeStruct((N, 1), jnp.float32),
    )(cnt, acc1, y1s, b1, W2)

    vs = jnp.pad(vs2d.reshape(1, N), ((0, 0), (0, ACC_ROWS - N)))
    acc2 = _make_scalar_seg_kernel(nchunk)(vs, srcp, dstp)

    out = pl.pallas_call(
        _tc3_body,
        grid=(grid,),
        in_specs=[
            pl.BlockSpec((NC, ACC_ROWS), lambda j: (0, 0)),
            pl.BlockSpec((NC, ACC_ROWS), lambda j: (0, 0)),
            pl.BlockSpec((_B, 1), lambda j: (j, 0)),
            pl.BlockSpec((1,), lambda j: (0,)),
        ],
        out_specs=pl.BlockSpec((_B, 1), lambda j: (j, 0)),
        out_shape=jax.ShapeDtypeStruct((N, 1), jnp.float32),
    )(cnt, acc2, vs2d, b2)

    return out
